# baseline (speedup 1.0000x reference)
import functools

import jax
import jax.numpy as jnp
from jax import lax
from jax.experimental import pallas as pl
from jax.experimental.pallas import tpu as pltpu
from jax.experimental.pallas import tpu_sc as plsc

_B = 4
_S = 2048
_D = 512
_NW = 32
_P = _S // _NW
_C = 32
_NCH = _B * _P // _C
_RING = 4
_NG = _D // 128


def _row_add(w, pos_v, r, pbase):
    def loads(g):
        return [pos_v[pbase + r, pl.ds((g * 8 + j) * 16, 16)]
                for j in range(8)]
    vals = loads(0)
    for g in range(_NG):
        nxt = loads(g + 1) if g + 1 < _NG else None
        for j in range(8):
            plsc.addupdate(w.at[r, pl.ds((g * 8 + j) * 16, 16)], vals[j])
        vals = nxt


def _emb_kernel(idx_hbm, word_hbm, pos_hbm, out_hbm,
                idx_v, pos_v, w0, w1, w2, w3,
                sg0, sg1, sg2, sg3, so0, so1, so2, so3, sp):
    wid = lax.axis_index("s") * 2 + lax.axis_index("c")
    pos_base = wid * _P
    pltpu.sync_copy(idx_hbm.at[:, wid], idx_v)
    pp = pltpu.async_copy(pos_hbm.at[pl.ds(pos_base, _P)], pos_v, sp)
    wbufs = (w0, w1, w2, w3)
    sgs, sos = (sg0, sg1, sg2, sg3), (so0, so1, so2, so3)

    def gather(c):
        b, h = c // 2, c % 2
        return pltpu.async_copy(
            word_hbm.at[idx_v.at[b, h]], wbufs[c % _RING], sgs[c % _RING])

    gs = [None] * _NCH
    outs = [None] * _NCH
    for c in range(_RING - 1):
        gs[c] = gather(c)
    pp.wait()
    for c in range(_NCH):
        k = c % _RING
        if c + _RING - 1 < _NCH:
            if outs[c - 1] is not None:
                outs[c - 1].wait()
            gs[c + _RING - 1] = gather(c + _RING - 1)
        gs[c].wait()
        w = wbufs[k]
        b, h = c // 2, c % 2

        def add_body(r, _, w=w, h=h):
            _row_add(w, pos_v, r, h * _C)
            return 0

        lax.fori_loop(0, _C, add_body, 0)
        outs[c] = pltpu.async_copy(
            w, out_hbm.at[pl.ds(b * _S + pos_base + h * _C, _C)], sos[k])
    for c in range(_NCH - _RING, _NCH):
        outs[c].wait()


@jax.jit
def _run(idx4d, word_emb, pos_emb):
    mesh = plsc.VectorSubcoreMesh(core_axis_name="c", subcore_axis_name="s")
    k = functools.partial(
        pl.kernel,
        mesh=mesh,
        out_type=jax.ShapeDtypeStruct((_B * _S, _D), jnp.float32),
        scratch_types=[
            pltpu.VMEM((_B, 2, _C), jnp.int32),
            pltpu.VMEM((_P, _D), jnp.float32),
            pltpu.VMEM((_C, _D), jnp.float32),
            pltpu.VMEM((_C, _D), jnp.float32),
            pltpu.VMEM((_C, _D), jnp.float32),
            pltpu.VMEM((_C, _D), jnp.float32),
            pltpu.SemaphoreType.DMA,
            pltpu.SemaphoreType.DMA,
            pltpu.SemaphoreType.DMA,
            pltpu.SemaphoreType.DMA,
            pltpu.SemaphoreType.DMA,
            pltpu.SemaphoreType.DMA,
            pltpu.SemaphoreType.DMA,
            pltpu.SemaphoreType.DMA,
            pltpu.SemaphoreType.DMA,
        ],
    )(_emb_kernel)
    return k(idx4d, word_emb, pos_emb)


def kernel(inputs, word_emb, pos_emb):
    idx4d = inputs.reshape(_B, _NW, 2, _C)
    out = _run(idx4d, word_emb, pos_emb)
    return out.reshape(_B, _S, _D)
